# Initial kernel scaffold; baseline (speedup 1.0000x reference)
import jax
import jax.numpy as jnp
from jax.experimental import pallas as pl


def kernel(x, W_enc, A, B, b_enc, b_dec):
    raise NotImplementedError("write your pallas kernel here")


# speedup vs baseline: 2.3032x; 2.3032x over previous
import jax
import jax.numpy as jnp
from jax import lax
from jax.experimental import pallas as pl
from jax.experimental.pallas import tpu as pltpu

B_, T_, D_IN, D_SAE, K_, R_ = 128, 5, 768, 16384, 64, 4
TD = T_ * D_IN
INT_MIN = -2147483648

ENC_SB = 1024
DEC_SB = 1024


def _enc_body(x_ref, w_ref, b_ref, out_ref):
    out_ref[...] = (
        jnp.dot(x_ref[...], w_ref[...], preferred_element_type=jnp.float32)
        + b_ref[...]
    )


def _topk_body(pre_ref, z_ref):
    pre = pre_ref[...]
    bi = lax.bitcast_convert_type(pre, jnp.int32)
    skey = jnp.where(pre < 0.0, (~bi) ^ jnp.int32(INT_MIN), bi)

    def vbit(b):
        return lax.shift_left(jnp.int32(1), b)

    def step_val(i, t_u):
        c_u = t_u | vbit(31 - i)
        cnt = jnp.sum((skey >= (c_u ^ jnp.int32(INT_MIN))).astype(jnp.int32),
                      axis=1, keepdims=True)
        return jnp.where(cnt >= K_, c_u, t_u)

    t_u = lax.fori_loop(0, 32, step_val, jnp.zeros((B_, 1), jnp.int32))
    tau = t_u ^ jnp.int32(INT_MIN)

    gt = skey > tau
    eq = skey == tau
    c1 = jnp.sum(gt.astype(jnp.int32), axis=1, keepdims=True)
    r = K_ - c1

    idx = lax.broadcasted_iota(jnp.int32, (B_, D_SAE), 1)

    def step_idx(i, m):
        c = m | vbit(13 - i)
        cnt = jnp.sum((eq & (idx < c)).astype(jnp.int32), axis=1,
                      keepdims=True)
        return jnp.where(cnt < r, c, m)

    m = lax.fori_loop(0, 14, step_idx, jnp.zeros((B_, 1), jnp.int32))
    mask = gt | (eq & (idx <= m))
    z_ref[...] = jnp.where(mask, jnp.maximum(pre, 0.0), 0.0)


def _dec_body(z_ref, a_ref, bm_ref, x_ref, bdec_ref, xhat_ref, loss_ref,
              acc_ref):
    i = pl.program_id(0)

    @pl.when(i == 0)
    def _():
        acc_ref[...] = jnp.zeros_like(acc_ref)

    z = z_ref[...]
    a = a_ref[...]
    bm = bm_ref[...]
    for t in range(T_):
        wd = a[:, t * R_ + 0][:, None] * bm[:, 0:D_IN]
        for rr in range(1, R_):
            wd += a[:, t * R_ + rr][:, None] * bm[:, rr * D_IN:(rr + 1) * D_IN]
        acc_ref[:, t * D_IN:(t + 1) * D_IN] += jnp.dot(
            z, wd, preferred_element_type=jnp.float32)

    @pl.when(i == pl.num_programs(0) - 1)
    def _():
        xh = acc_ref[...] + bdec_ref[...]
        xhat_ref[...] = xh
        d = xh - x_ref[...]
        loss_ref[...] = jnp.broadcast_to(jnp.sum(d * d) / (B_ * T_), (1, 1))


def kernel(x, W_enc, A, B, b_enc, b_dec):
    x2 = x.reshape(B_, TD)
    W2 = W_enc.reshape(TD, D_SAE)
    b_enc2 = b_enc.reshape(1, D_SAE)
    A2 = A.reshape(D_SAE, T_ * R_)
    B2 = B.reshape(D_SAE, R_ * D_IN)
    b_dec2 = b_dec.reshape(1, TD)

    pre = pl.pallas_call(
        _enc_body,
        grid=(D_SAE // ENC_SB,),
        in_specs=[
            pl.BlockSpec((B_, TD), lambda i: (0, 0)),
            pl.BlockSpec((TD, ENC_SB), lambda i: (0, i)),
            pl.BlockSpec((1, ENC_SB), lambda i: (0, i)),
        ],
        out_specs=pl.BlockSpec((B_, ENC_SB), lambda i: (0, i)),
        out_shape=jax.ShapeDtypeStruct((B_, D_SAE), jnp.float32),
        compiler_params=pltpu.CompilerParams(
            dimension_semantics=("parallel",)),
    )(x2, W2, b_enc2)

    z = pl.pallas_call(
        _topk_body,
        in_specs=[pl.BlockSpec((B_, D_SAE), lambda: (0, 0))],
        out_specs=pl.BlockSpec((B_, D_SAE), lambda: (0, 0)),
        out_shape=jax.ShapeDtypeStruct((B_, D_SAE), jnp.float32),
    )(pre)

    xhat2, loss2 = pl.pallas_call(
        _dec_body,
        grid=(D_SAE // DEC_SB,),
        in_specs=[
            pl.BlockSpec((B_, DEC_SB), lambda i: (0, i)),
            pl.BlockSpec((DEC_SB, T_ * R_), lambda i: (i, 0)),
            pl.BlockSpec((DEC_SB, R_ * D_IN), lambda i: (i, 0)),
            pl.BlockSpec((B_, TD), lambda i: (0, 0)),
            pl.BlockSpec((1, TD), lambda i: (0, 0)),
        ],
        out_specs=[
            pl.BlockSpec((B_, TD), lambda i: (0, 0)),
            pl.BlockSpec((1, 1), lambda i: (0, 0)),
        ],
        out_shape=[
            jax.ShapeDtypeStruct((B_, TD), jnp.float32),
            jax.ShapeDtypeStruct((1, 1), jnp.float32),
        ],
        scratch_shapes=[pltpu.VMEM((B_, TD), jnp.float32)],
        compiler_params=pltpu.CompilerParams(
            dimension_semantics=("arbitrary",)),
    )(z, A2, B2, x2, b_dec2)

    loss = loss2[0, 0]
    x_hat = xhat2.reshape(B_, T_, D_IN)
    return (loss, x_hat, z)
